# baseline (device time: 19203 ns/iter reference)
import jax
import jax.numpy as jnp
from jax import lax
from jax.experimental import pallas as pl
from jax.experimental.pallas import tpu as pltpu

N_DEV = 16
E_LOCAL = 4
N_ROWS = 512
ROWS_PER = N_ROWS // N_DEV
D_MODEL = 256
D_OUT = 512
N_EXPERTS = 64


def kernel(x, router_W, route_idx, expert_W, shared_W):
    def body(x_ref, rw_ref, idx_ref, ew_ref, sw_ref, out_ref,
             partial_ref, recv_ref, send_sems, recv_sems):
        my = lax.axis_index("i")

        barrier_sem = pltpu.get_barrier_semaphore()
        for k in range(1, N_DEV):
            peer = (my + k) % N_DEV
            pl.semaphore_signal(
                barrier_sem, inc=1,
                device_id=(peer,), device_id_type=pl.DeviceIdType.MESH,
            )
        pl.semaphore_wait(barrier_sem, N_DEV - 1)

        xf = x_ref[...]
        scores = jnp.dot(xf, rw_ref[...], preferred_element_type=jnp.float32)
        scores = scores - jnp.max(scores, axis=1, keepdims=True)
        ex = jnp.exp(scores)
        probs = ex / jnp.sum(ex, axis=1, keepdims=True)
        idx = idx_ref[...]
        eids = lax.broadcasted_iota(jnp.int32, (N_ROWS, N_EXPERTS), 1)
        probs_sel = jnp.sum(
            jnp.where(eids == idx, probs, 0.0), axis=1, keepdims=True
        )

        part = jnp.zeros((N_ROWS, D_OUT), jnp.float32)
        for le in range(E_LOCAL):
            e = my * E_LOCAL + le
            coeff = jnp.where(idx == e, probs_sel, 0.0)
            xs = (xf * coeff).astype(jnp.bfloat16)
            w = ew_ref[le].astype(jnp.bfloat16)
            part = part + jnp.dot(xs, w, preferred_element_type=jnp.float32)
        partial_ref[...] = part.astype(jnp.bfloat16)

        sends = []
        for k in range(1, N_DEV):
            dst = (my + k) % N_DEV
            rdma = pltpu.make_async_remote_copy(
                src_ref=partial_ref.at[pl.ds(dst * ROWS_PER, ROWS_PER)],
                dst_ref=recv_ref.at[my],
                send_sem=send_sems.at[dst],
                recv_sem=recv_sems.at[my],
                device_id=(dst,),
                device_id_type=pl.DeviceIdType.MESH,
            )
            rdma.start()
            sends.append(rdma)

        x_own = x_ref[pl.ds(my * ROWS_PER, ROWS_PER), :]
        shared = jnp.dot(
            x_own.astype(jnp.bfloat16), sw_ref[...].astype(jnp.bfloat16),
            preferred_element_type=jnp.float32,
        )
        acc = shared + partial_ref[pl.ds(my * ROWS_PER, ROWS_PER), :].astype(
            jnp.float32
        )

        for k in range(1, N_DEV):
            src = (my + k) % N_DEV
            recv = pltpu.make_async_remote_copy(
                src_ref=partial_ref.at[pl.ds(0, ROWS_PER)],
                dst_ref=recv_ref.at[src],
                send_sem=send_sems.at[src],
                recv_sem=recv_sems.at[src],
                device_id=(src,),
                device_id_type=pl.DeviceIdType.MESH,
            )
            recv.wait_recv()
            acc = acc + recv_ref[src].astype(jnp.float32)

        out_ref[...] = acc

        for rdma in sends:
            rdma.wait_send()

    return pl.pallas_call(
        body,
        out_shape=jax.ShapeDtypeStruct((ROWS_PER, D_OUT), jnp.float32),
        in_specs=[pl.BlockSpec(memory_space=pltpu.VMEM)] * 5,
        out_specs=pl.BlockSpec(memory_space=pltpu.VMEM),
        scratch_shapes=[
            pltpu.VMEM((N_ROWS, D_OUT), jnp.bfloat16),
            pltpu.VMEM((N_DEV, ROWS_PER, D_OUT), jnp.bfloat16),
            pltpu.SemaphoreType.DMA((N_DEV,)),
            pltpu.SemaphoreType.DMA((N_DEV,)),
        ],
        compiler_params=pltpu.CompilerParams(collective_id=0),
    )(x, router_W, route_idx, expert_W, shared_W)
